# Initial kernel scaffold; baseline (speedup 1.0000x reference)
import jax
import jax.numpy as jnp
from jax.experimental import pallas as pl


def kernel(pred_ratios, demands, neighbor_loads, link_capacities, tunnel_to_link, group_indices):
    raise NotImplementedError("write your pallas kernel here")


# speedup vs baseline: 1.3378x; 1.3378x over previous
import functools

import jax
import jax.numpy as jnp
from jax import lax
from jax.experimental import pallas as pl
from jax.experimental.pallas import tpu as pltpu
from jax.experimental.pallas import tpu_sc as plsc

B = 4096
D = 64
K = 32
T = D * K
L = 64

NC = 2
NS = 16
NW = NC * NS
RPW = B // NW
RB = 8
NG = RPW // RB
LCH = L // 16


def _sc_body(pred_hbm, dem_hbm, nb_hbm, cap_hbm, t2l_hbm, out_hbm,
             t2l_v, cap_v, dem_v, nb_v, pred_v, acc_v, out_v, sem):
    cid = lax.axis_index("c")
    sid = lax.axis_index("s")
    wid = sid * NC + cid
    base = wid * RPW

    pltpu.sync_copy(t2l_hbm, t2l_v)
    pltpu.sync_copy(cap_hbm, cap_v)
    pltpu.sync_copy(dem_hbm.at[pl.ds(base, RPW)], dem_v)
    pltpu.sync_copy(nb_hbm.at[pl.ds(base, RPW)], nb_v)

    inv_cap = [1.0 / (cap_v[pl.ds(16 * j, 16)] + 1e-8) for j in range(LCH)]
    zeros16 = jnp.zeros((16,), jnp.float32)

    def block_body(g, carry):
        vs, ms, cs_vec = carry
        pltpu.sync_copy(pred_hbm.at[pl.ds(base + g * RB, RB)], pred_v)
        for r in range(RB):
            row = g * RB + r
            for j in range(LCH):
                acc_v[pl.ds(16 * j, 16)] = zeros16

            row_splat = jnp.full((16,), row, jnp.int32)

            def chunk_body(d, _):
                dsplat = plsc.load_gather(
                    dem_v, [row_splat, jnp.full((16,), d, jnp.int32)])
                for h in range(2):
                    off = 32 * d + 16 * h
                    links = t2l_v[pl.ds(off, 16)]
                    tt = pred_v[r, pl.ds(off, 16)] * dsplat
                    plsc.addupdate_scatter(acc_v, [links], tt)
                return 0

            lax.fori_loop(0, D, chunk_body, 0)

            lt = [acc_v[pl.ds(16 * j, 16)] for j in range(LCH)]
            u = [lt[j] * inv_cap[j] for j in range(LCH)]
            nb = [nb_v[row, pl.ds(16 * j, 16)] for j in range(LCH)]
            usum = (u[0] + u[1]) + (u[2] + u[3])
            usq = (u[0] * u[0] + u[1] * u[1]) + (u[2] * u[2] + u[3] * u[3])
            umax = jnp.maximum(jnp.maximum(u[0], u[1]), jnp.maximum(u[2], u[3]))
            ltsum = (lt[0] + lt[1]) + (lt[2] + lt[3])
            nbsum = (nb[0] + nb[1]) + (nb[2] + nb[3])
            dot = (lt[0] * nb[0] + lt[1] * nb[1]) + (lt[2] * nb[2] + lt[3] * nb[3])

            s = jnp.sum(usum)
            q = jnp.sum(usq)
            m = jnp.max(umax)
            ts = jnp.sum(ltsum)
            ns = jnp.sum(nbsum)
            dp = jnp.sum(dot)

            vs = vs + (q - s * s * (1.0 / L)) * (1.0 / (L - 1))
            ms = ms + m
            denom = (ts + 1e-8) * (ns + 1e-8)
            cs_vec = cs_vec + jnp.full((16,), dp) / jnp.full((16,), denom)
        return vs, ms, cs_vec

    vs, ms, cs_vec = lax.fori_loop(
        0, NG, block_body,
        (jnp.float32(0.0), jnp.float32(0.0), jnp.zeros((16,), jnp.float32)))

    lane = lax.iota(jnp.int32, 16)
    vec = jnp.where(lane == 0, jnp.full((16,), vs),
                    jnp.where(lane == 1, jnp.full((16,), ms),
                              jnp.where(lane == 2, cs_vec,
                                        jnp.zeros((16,), jnp.float32))))
    out_v[...] = vec
    pltpu.sync_copy(out_v, out_hbm.at[wid])


@jax.jit
def _sc_loss(pred_ratios, demands, neighbor_loads, link_capacities, tunnel_to_link):
    mesh = plsc.VectorSubcoreMesh(core_axis_name="c", subcore_axis_name="s")
    partials = pl.kernel(
        _sc_body,
        out_type=jax.ShapeDtypeStruct((NW, 16), jnp.float32),
        mesh=mesh,
        compiler_params=pltpu.CompilerParams(needs_layout_passes=False),
        scratch_types=[
            pltpu.VMEM((T,), jnp.int32),
            pltpu.VMEM((L,), jnp.float32),
            pltpu.VMEM((RPW, D), jnp.float32),
            pltpu.VMEM((RPW, L), jnp.float32),
            pltpu.VMEM((RB, T), jnp.float32),
            pltpu.VMEM((L,), jnp.float32),
            pltpu.VMEM((16,), jnp.float32),
            pltpu.SemaphoreType.DMA,
        ],
    )(pred_ratios, demands, neighbor_loads, link_capacities, tunnel_to_link)
    vs = jnp.sum(partials[:, 0])
    ms = jnp.sum(partials[:, 1])
    cs = jnp.sum(partials[:, 2])
    return (vs + 0.5 * ms + 0.3 * cs) * (1.0 / B)


def kernel(pred_ratios, demands, neighbor_loads, link_capacities, tunnel_to_link, group_indices):
    del group_indices
    return _sc_loss(pred_ratios, demands, neighbor_loads, link_capacities,
                    tunnel_to_link)
